# Initial kernel scaffold; baseline (speedup 1.0000x reference)
import jax
import jax.numpy as jnp
from jax.experimental import pallas as pl


def kernel(x, edge_index, A_values, gamma1, beta1, mean1, var1, w1, b1, c1, gamma2, beta2, mean2, var2, w2, b2, c2):
    raise NotImplementedError("write your pallas kernel here")


# speedup vs baseline: 3.3277x; 3.3277x over previous
import functools

import jax
import jax.numpy as jnp
from jax import lax
from jax.experimental import pallas as pl
from jax.experimental.pallas import tpu as pltpu
from jax.experimental.pallas import tpu_sc as plsc

N, D, E = 10000, 256, 160000
DH = D // 2
NS = 16
K = 128
CHUNKS = E // K
CPT = CHUNKS // NS
RPT = 624
TAIL = N - RPT * NS
ZR = 208

_mesh = plsc.VectorSubcoreMesh(core_axis_name="c", subcore_axis_name="s")


def _spmm_body(src_hbm, dst_hbm, val_hbm, h_lo, h_hi, out_lo, out_hi,
               acc, src_v, dst_v, val_v, rows_v, zero_v, gsem):
    c = lax.axis_index("c")
    s = lax.axis_index("s")

    zeros16 = jnp.zeros((16,), jnp.float32)

    def _zrow(r, carry):
        for cc in range(DH // 16):
            zero_v[r, pl.ds(cc * 16, 16)] = zeros16
        return carry

    lax.fori_loop(0, ZR, _zrow, 0)
    for j in range(RPT // ZR):
        pltpu.sync_copy(zero_v, acc.at[pl.ds(s * RPT + j * ZR, ZR)])

    @pl.when(s == 0)
    def _():
        pltpu.sync_copy(zero_v.at[pl.ds(0, TAIL)], acc.at[pl.ds(RPT * NS, TAIL)])

    plsc.subcore_barrier()

    def _chunk(i, carry):
        cidx = s + i * NS
        base = cidx * K
        pltpu.sync_copy(src_hbm.at[pl.ds(base, K)], src_v)
        pltpu.sync_copy(dst_hbm.at[pl.ds(base, K)], dst_v)
        pltpu.sync_copy(val_hbm.at[pl.ds(base, K)], val_v)

        @pl.when(c == 0)
        def _():
            pltpu.async_copy(h_lo.at[src_v], rows_v, gsem).wait()

        @pl.when(c == 1)
        def _():
            pltpu.async_copy(h_hi.at[src_v], rows_v, gsem).wait()

        def _group(g, carry2):
            v16 = val_v[pl.ds(g * 16, 16)]
            for j in range(16):
                e = g * 16 + j
                vj = v16[j]
                for cc in range(DH // 16):
                    sl = pl.ds(cc * 16, 16)
                    rows_v[e, sl] = rows_v[e, sl] * vj
            return carry2

        lax.fori_loop(0, K // 16, _group, 0)
        pltpu.sync_copy(rows_v, acc.at[dst_v], add=True)
        return carry

    nchunks = CPT + jnp.where(s < CHUNKS - CPT * NS, 1, 0)
    lax.fori_loop(0, nchunks, _chunk, 0)
    plsc.subcore_barrier()

    @pl.when(c == 0)
    def _():
        pltpu.sync_copy(acc.at[pl.ds(s * RPT, RPT)], out_lo.at[pl.ds(s * RPT, RPT)])

        @pl.when(s == 0)
        def _():
            pltpu.sync_copy(acc.at[pl.ds(RPT * NS, TAIL)],
                            out_lo.at[pl.ds(RPT * NS, TAIL)])

    @pl.when(c == 1)
    def _():
        pltpu.sync_copy(acc.at[pl.ds(s * RPT, RPT)], out_hi.at[pl.ds(s * RPT, RPT)])

        @pl.when(s == 0)
        def _():
            pltpu.sync_copy(acc.at[pl.ds(RPT * NS, TAIL)],
                            out_hi.at[pl.ds(RPT * NS, TAIL)])


_spmm = pl.kernel(
    _spmm_body,
    out_type=(jax.ShapeDtypeStruct((N, DH), jnp.float32),
              jax.ShapeDtypeStruct((N, DH), jnp.float32)),
    mesh=_mesh,
    scratch_types=[
        pltpu.VMEM_SHARED((N, DH), jnp.float32),
        pltpu.VMEM((K,), jnp.int32),
        pltpu.VMEM((K,), jnp.int32),
        pltpu.VMEM((K,), jnp.float32),
        pltpu.VMEM((K, DH), jnp.float32),
        pltpu.VMEM((ZR, DH), jnp.float32),
        pltpu.SemaphoreType.DMA,
    ],
)


RB = 1000
GRID = N // RB


def _l2n(x):
    sq = jnp.sum(x * x, axis=1, keepdims=True)
    return x * lax.rsqrt(jnp.maximum(sq, 1e-12))


def _pre_body(x_ref, sc_ref, sh_ref, hlo_ref, hhi_ref, xn_ref):
    x = x_ref[...]
    h = x * sc_ref[...] + sh_ref[...]
    hlo_ref[...] = h[:, :DH]
    hhi_ref[...] = h[:, DH:]
    xn_ref[...] = _l2n(x)


def _mid_body(axlo_ref, axhi_ref, hlo_ref, hhi_ref, w_ref, b_ref, c_ref,
              sc_ref, sh_ref, yn_ref, h2lo_ref, h2hi_ref):
    ax = jnp.concatenate([axlo_ref[...], axhi_ref[...]], axis=1)
    h = jnp.concatenate([hlo_ref[...], hhi_ref[...]], axis=1)
    z = ax + h * (c_ref[0, 0] + 1.0)
    y = jnp.tanh(jnp.dot(z, w_ref[...], preferred_element_type=jnp.float32)
                 + b_ref[...])
    yn_ref[...] = _l2n(y)
    h2 = y * sc_ref[...] + sh_ref[...]
    h2lo_ref[...] = h2[:, :DH]
    h2hi_ref[...] = h2[:, DH:]


def _post_body(axlo_ref, axhi_ref, hlo_ref, hhi_ref, w_ref, b_ref, c_ref,
               xn_ref, y1n_ref, out_ref):
    ax = jnp.concatenate([axlo_ref[...], axhi_ref[...]], axis=1)
    h = jnp.concatenate([hlo_ref[...], hhi_ref[...]], axis=1)
    z = ax + h * (c_ref[0, 0] + 1.0)
    y2 = jnp.tanh(jnp.dot(z, w_ref[...], preferred_element_type=jnp.float32)
                  + b_ref[...])
    y2n = _l2n(y2)
    xn = xn_ref[...]
    y1n = y1n_ref[...]
    ssum = (jnp.sum(xn * xn, axis=1, keepdims=True)
            + jnp.sum(y1n * y1n, axis=1, keepdims=True)
            + jnp.sum(y2n * y2n, axis=1, keepdims=True))
    r = lax.rsqrt(jnp.maximum(ssum, 1e-12))
    out_ref[:, :D] = xn * r
    out_ref[:, D:2 * D] = y1n * r
    out_ref[:, 2 * D:] = y2n * r


def _row_spec(w):
    return pl.BlockSpec((RB, w), lambda i: (i, 0))


def _rep_spec(shape):
    return pl.BlockSpec(shape, lambda i: tuple(0 for _ in shape))


_smem_spec = pl.BlockSpec(memory_space=pltpu.SMEM)

_pre = pl.pallas_call(
    _pre_body,
    grid=(GRID,),
    in_specs=[_row_spec(D), _rep_spec((1, D)), _rep_spec((1, D))],
    out_specs=(_row_spec(DH), _row_spec(DH), _row_spec(D)),
    out_shape=(jax.ShapeDtypeStruct((N, DH), jnp.float32),
               jax.ShapeDtypeStruct((N, DH), jnp.float32),
               jax.ShapeDtypeStruct((N, D), jnp.float32)),
)

_mid = pl.pallas_call(
    _mid_body,
    grid=(GRID,),
    in_specs=[_row_spec(DH), _row_spec(DH), _row_spec(DH), _row_spec(DH),
              _rep_spec((D, D)), _rep_spec((1, D)), _smem_spec,
              _rep_spec((1, D)), _rep_spec((1, D))],
    out_specs=(_row_spec(D), _row_spec(DH), _row_spec(DH)),
    out_shape=(jax.ShapeDtypeStruct((N, D), jnp.float32),
               jax.ShapeDtypeStruct((N, DH), jnp.float32),
               jax.ShapeDtypeStruct((N, DH), jnp.float32)),
)

_post = pl.pallas_call(
    _post_body,
    grid=(GRID,),
    in_specs=[_row_spec(DH), _row_spec(DH), _row_spec(DH), _row_spec(DH),
              _rep_spec((D, D)), _rep_spec((1, D)), _smem_spec,
              _row_spec(D), _row_spec(D)],
    out_specs=_row_spec(3 * D),
    out_shape=jax.ShapeDtypeStruct((N, 3 * D), jnp.float32),
)


@jax.jit
def kernel(x, edge_index, A_values, gamma1, beta1, mean1, var1, w1, b1, c1,
           gamma2, beta2, mean2, var2, w2, b2, c2):
    src = edge_index[0]
    dst = edge_index[1]

    def _affine(gamma, beta, mean, var):
        s = gamma * lax.rsqrt(var + 1e-3)
        return s.reshape(1, D), (beta - mean * s).reshape(1, D)

    sc1, sh1 = _affine(gamma1, beta1, mean1, var1)
    sc2, sh2 = _affine(gamma2, beta2, mean2, var2)

    h1_lo, h1_hi, xn = _pre(x, sc1, sh1)
    ax1_lo, ax1_hi = _spmm(src, dst, A_values, h1_lo, h1_hi)
    y1n, h2_lo, h2_hi = _mid(ax1_lo, ax1_hi, h1_lo, h1_hi, w1,
                             b1.reshape(1, D), c1, sc2, sh2)
    ax2_lo, ax2_hi = _spmm(src, dst, A_values, h2_lo, h2_hi)
    out = _post(ax2_lo, ax2_hi, h2_lo, h2_hi, w2, b2.reshape(1, D), c2,
                xn, y1n)
    return out


# speedup vs baseline: 6.2464x; 1.8771x over previous
import functools

import jax
import jax.numpy as jnp
from jax import lax
from jax.experimental import pallas as pl
from jax.experimental.pallas import tpu as pltpu
from jax.experimental.pallas import tpu_sc as plsc

N, D, E = 10000, 256, 160000
DH = D // 2
NS = 16
K = 128
CHUNKS = E // K
CPT = CHUNKS // NS
RPT = 624
TAIL = N - RPT * NS

_mesh = plsc.VectorSubcoreMesh(core_axis_name="c", subcore_axis_name="s")


def _spmm_body(epk_hbm, val_hbm, h3, out3,
               acc, idx0, idx1, val0, val1, rows0, rows1,
               isem0, isem1, gsem0, gsem1):
    c = lax.axis_index("c")
    s = lax.axis_index("s")
    hsel = h3.at[c]
    idxb = (idx0, idx1)
    valb = (val0, val1)
    rowsb = (rows0, rows1)
    isem = (isem0, isem1)
    gsem = (gsem0, gsem1)

    zeros16 = jnp.zeros((16,), jnp.float32)

    def _zrow(r, carry):
        for cc in range(DH // 16):
            rows0[r, pl.ds(cc * 16, 16)] = zeros16
        return carry

    lax.fori_loop(0, K, _zrow, 0)
    for j in range(4):
        pltpu.sync_copy(rows0, acc.at[pl.ds(s * RPT + j * K, K)])
    pltpu.sync_copy(rows0.at[pl.ds(0, RPT - 4 * K)],
                    acc.at[pl.ds(s * RPT + 4 * K, RPT - 4 * K)])

    @pl.when(s == 0)
    def _():
        pltpu.sync_copy(rows0.at[pl.ds(0, TAIL)], acc.at[pl.ds(RPT * NS, TAIL)])

    plsc.subcore_barrier()

    def _base(chunk):
        return (s + chunk * NS) * K

    def idx_start(b, chunk):
        base = _base(chunk)
        pltpu.async_copy(epk_hbm.at[:, pl.ds(base, K)], idxb[b], isem[b])
        pltpu.async_copy(val_hbm.at[pl.ds(base, K)], valb[b], isem[b])

    def idx_wait(b):
        pltpu.make_async_copy(epk_hbm.at[:, pl.ds(0, K)], idxb[b], isem[b]).wait()
        pltpu.make_async_copy(val_hbm.at[pl.ds(0, K)], valb[b], isem[b]).wait()

    def gather_start(b):
        pltpu.async_copy(hsel.at[idxb[b].at[0]], rowsb[b], gsem[b])

    def gather_wait(b):
        pltpu.make_async_copy(hsel.at[idxb[b].at[0]], rowsb[b], gsem[b]).wait()

    def compute(b):
        def _group(g, carry):
            v16 = valb[b][pl.ds(g * 16, 16)]
            for j in range(16):
                e = g * 16 + j
                vj = v16[j]
                for cc in range(DH // 16):
                    sl = pl.ds(cc * 16, 16)
                    rowsb[b][e, sl] = rowsb[b][e, sl] * vj
            return carry

        lax.fori_loop(0, K // 16, _group, 0)

    def scat(b):
        pltpu.sync_copy(rowsb[b], acc.at[idxb[b].at[1]], add=True)

    idx_start(0, 0)
    idx_wait(0)
    gather_start(0)
    idx_start(1, 1)

    def _pair(j, carry):
        for b in (0, 1):
            chunk = 2 * j + b
            nb = 1 - b

            @pl.when(chunk < CPT - 1)
            def _():
                idx_wait(nb)
                gather_start(nb)

            gather_wait(b)
            compute(b)
            scat(b)

            @pl.when(chunk < CPT - 2)
            def _():
                idx_start(b, chunk + 2)
        return carry

    lax.fori_loop(0, CPT // 2, _pair, 0)

    @pl.when(s < CHUNKS - CPT * NS)
    def _():
        idx_start(0, CPT)
        idx_wait(0)
        gather_start(0)
        gather_wait(0)
        compute(0)
        scat(0)

    plsc.subcore_barrier()

    osel = out3.at[c]
    pltpu.sync_copy(acc.at[pl.ds(s * RPT, RPT)], osel.at[pl.ds(s * RPT, RPT)])

    @pl.when(s == 0)
    def _():
        pltpu.sync_copy(acc.at[pl.ds(RPT * NS, TAIL)],
                        osel.at[pl.ds(RPT * NS, TAIL)])


_spmm = pl.kernel(
    _spmm_body,
    out_type=jax.ShapeDtypeStruct((2, N, DH), jnp.float32),
    mesh=_mesh,
    scratch_types=[
        pltpu.VMEM_SHARED((N, DH), jnp.float32),
        pltpu.VMEM((2, K), jnp.int32),
        pltpu.VMEM((2, K), jnp.int32),
        pltpu.VMEM((K,), jnp.float32),
        pltpu.VMEM((K,), jnp.float32),
        pltpu.VMEM((K, DH), jnp.float32),
        pltpu.VMEM((K, DH), jnp.float32),
        pltpu.SemaphoreType.DMA,
        pltpu.SemaphoreType.DMA,
        pltpu.SemaphoreType.DMA,
        pltpu.SemaphoreType.DMA,
    ],
)


RB = 1000
GRID = N // RB


def _l2n(x):
    sq = jnp.sum(x * x, axis=1, keepdims=True)
    return x * lax.rsqrt(jnp.maximum(sq, 1e-12))


def _pre_body(x_ref, sc_ref, sh_ref, h3_ref, xn_ref):
    x = x_ref[...]
    h = x * sc_ref[...] + sh_ref[...]
    h3_ref[0] = h[:, :DH]
    h3_ref[1] = h[:, DH:]
    xn_ref[...] = _l2n(x)


def _mid_body(ax3_ref, h3_ref, w_ref, b_ref, c_ref,
              sc_ref, sh_ref, yn_ref, h23_ref):
    ax = jnp.concatenate([ax3_ref[0], ax3_ref[1]], axis=1)
    h = jnp.concatenate([h3_ref[0], h3_ref[1]], axis=1)
    z = ax + h * (c_ref[0, 0] + 1.0)
    y = jnp.tanh(jnp.dot(z, w_ref[...], preferred_element_type=jnp.float32)
                 + b_ref[...])
    yn_ref[...] = _l2n(y)
    h2 = y * sc_ref[...] + sh_ref[...]
    h23_ref[0] = h2[:, :DH]
    h23_ref[1] = h2[:, DH:]


def _post_body(ax3_ref, h3_ref, w_ref, b_ref, c_ref,
               xn_ref, y1n_ref, out_ref):
    ax = jnp.concatenate([ax3_ref[0], ax3_ref[1]], axis=1)
    h = jnp.concatenate([h3_ref[0], h3_ref[1]], axis=1)
    z = ax + h * (c_ref[0, 0] + 1.0)
    y2 = jnp.tanh(jnp.dot(z, w_ref[...], preferred_element_type=jnp.float32)
                  + b_ref[...])
    y2n = _l2n(y2)
    xn = xn_ref[...]
    y1n = y1n_ref[...]
    ssum = (jnp.sum(xn * xn, axis=1, keepdims=True)
            + jnp.sum(y1n * y1n, axis=1, keepdims=True)
            + jnp.sum(y2n * y2n, axis=1, keepdims=True))
    r = lax.rsqrt(jnp.maximum(ssum, 1e-12))
    out_ref[:, :D] = xn * r
    out_ref[:, D:2 * D] = y1n * r
    out_ref[:, 2 * D:] = y2n * r


def _row_spec(w):
    return pl.BlockSpec((RB, w), lambda i: (i, 0))


def _half_spec():
    return pl.BlockSpec((2, RB, DH), lambda i: (0, i, 0))


def _rep_spec(shape):
    return pl.BlockSpec(shape, lambda i: tuple(0 for _ in shape))


_smem_spec = pl.BlockSpec(memory_space=pltpu.SMEM)

_h3_shape = jax.ShapeDtypeStruct((2, N, DH), jnp.float32)

_pre = pl.pallas_call(
    _pre_body,
    grid=(GRID,),
    in_specs=[_row_spec(D), _rep_spec((1, D)), _rep_spec((1, D))],
    out_specs=(_half_spec(), _row_spec(D)),
    out_shape=(_h3_shape, jax.ShapeDtypeStruct((N, D), jnp.float32)),
)

_mid = pl.pallas_call(
    _mid_body,
    grid=(GRID,),
    in_specs=[_half_spec(), _half_spec(),
              _rep_spec((D, D)), _rep_spec((1, D)), _smem_spec,
              _rep_spec((1, D)), _rep_spec((1, D))],
    out_specs=(_row_spec(D), _half_spec()),
    out_shape=(jax.ShapeDtypeStruct((N, D), jnp.float32), _h3_shape),
)

_post = pl.pallas_call(
    _post_body,
    grid=(GRID,),
    in_specs=[_half_spec(), _half_spec(),
              _rep_spec((D, D)), _rep_spec((1, D)), _smem_spec,
              _row_spec(D), _row_spec(D)],
    out_specs=_row_spec(3 * D),
    out_shape=jax.ShapeDtypeStruct((N, 3 * D), jnp.float32),
)


@jax.jit
def kernel(x, edge_index, A_values, gamma1, beta1, mean1, var1, w1, b1, c1,
           gamma2, beta2, mean2, var2, w2, b2, c2):
    epk = edge_index

    def _affine(gamma, beta, mean, var):
        s = gamma * lax.rsqrt(var + 1e-3)
        return s.reshape(1, D), (beta - mean * s).reshape(1, D)

    sc1, sh1 = _affine(gamma1, beta1, mean1, var1)
    sc2, sh2 = _affine(gamma2, beta2, mean2, var2)

    h1, xn = _pre(x, sc1, sh1)
    ax1 = _spmm(epk, A_values, h1)
    y1n, h2 = _mid(ax1, h1, w1, b1.reshape(1, D), c1, sc2, sh2)
    ax2 = _spmm(epk, A_values, h2)
    out = _post(ax2, h2, w2, b2.reshape(1, D), c2, xn, y1n)
    return out


# speedup vs baseline: 7.0467x; 1.1281x over previous
import functools

import jax
import jax.numpy as jnp
from jax import lax
from jax.experimental import pallas as pl
from jax.experimental.pallas import tpu as pltpu
from jax.experimental.pallas import tpu_sc as plsc

N, D, E = 10000, 256, 160000
DH = D // 2
NS = 16
K = 128
CHUNKS = E // K
CPT = CHUNKS // NS
RPT = 624
TAIL = N - RPT * NS

_mesh = plsc.VectorSubcoreMesh(core_axis_name="c", subcore_axis_name="s")


def _spmm_body(epk_hbm, val_hbm, h3, out3,
               acc, idx0, idx1, val0, val1, rows0, rows1, rows2,
               dst0, dst1, dst2,
               isem0, isem1, gsem0, gsem1, gsem2, ssem0, ssem1, ssem2):
    c = lax.axis_index("c")
    s = lax.axis_index("s")
    hsel = h3.at[c]
    idxb = (idx0, idx1)
    valb = (val0, val1)
    rowsb = (rows0, rows1, rows2)
    dstb = (dst0, dst1, dst2)
    isem = (isem0, isem1)
    gsem = (gsem0, gsem1, gsem2)
    ssem = (ssem0, ssem1, ssem2)

    zeros16 = jnp.zeros((16,), jnp.float32)

    def _zrow(r, carry):
        for cc in range(DH // 16):
            rows0[r, pl.ds(cc * 16, 16)] = zeros16
        return carry

    lax.fori_loop(0, K, _zrow, 0)
    for j in range(4):
        pltpu.sync_copy(rows0, acc.at[pl.ds(s * RPT + j * K, K)])
    pltpu.sync_copy(rows0.at[pl.ds(0, RPT - 4 * K)],
                    acc.at[pl.ds(s * RPT + 4 * K, RPT - 4 * K)])

    @pl.when(s == 0)
    def _():
        pltpu.sync_copy(rows0.at[pl.ds(0, TAIL)], acc.at[pl.ds(RPT * NS, TAIL)])

    plsc.subcore_barrier()

    def _base(chunk):
        return (s + chunk * NS) * K

    def idx_start(q, chunk):
        base = _base(chunk)
        pltpu.async_copy(epk_hbm.at[:, pl.ds(base, K)], idxb[q], isem[q])
        pltpu.async_copy(val_hbm.at[pl.ds(base, K)], valb[q], isem[q])

    def idx_wait(q):
        pltpu.make_async_copy(epk_hbm.at[:, pl.ds(0, K)], idxb[q], isem[q]).wait()
        pltpu.make_async_copy(val_hbm.at[pl.ds(0, K)], valb[q], isem[q]).wait()

    def gather_start(q, r):
        pltpu.async_copy(hsel.at[idxb[q].at[0]], rowsb[r], gsem[r])

    def gather_wait(q, r):
        pltpu.make_async_copy(hsel.at[idxb[q].at[0]], rowsb[r], gsem[r]).wait()

    def compute(q, r):
        def _group(g, carry):
            sl16 = pl.ds(g * 16, 16)
            dstb[r][sl16] = idxb[q][1, sl16]
            v16 = valb[q][sl16]
            for j in range(16):
                e = g * 16 + j
                vj = v16[j]
                for cc in range(DH // 16):
                    sl = pl.ds(cc * 16, 16)
                    rowsb[r][e, sl] = rowsb[r][e, sl] * vj
            return carry

        lax.fori_loop(0, K // 16, _group, 0)

    def scat_start(r):
        pltpu.async_copy(rowsb[r], acc.at[dstb[r]], ssem[r], add=True)

    def scat_wait(r):
        pltpu.make_async_copy(rowsb[r], acc.at[dstb[r]], ssem[r]).wait()

    idx_start(0, 0)
    idx_wait(0)
    gather_start(0, 0)
    idx_start(1, 1)

    UNROLL = 6

    def _macro(m, carry):
        for u in range(UNROLL):
            i = m * UNROLL + u
            q = u % 2
            r = u % 3
            nq = (u + 1) % 2
            nr = (u + 1) % 3
            chunk = i

            gather_wait(q, r)

            @pl.when(chunk < CPT - 1)
            def _():
                idx_wait(nq)
                @pl.when(chunk >= 2)
                def _():
                    scat_wait(nr)
                gather_start(nq, nr)

            compute(q, r)
            scat_start(r)

            @pl.when(chunk < CPT - 2)
            def _():
                idx_start(q, chunk + 2)
        return carry

    lax.fori_loop(0, CPT // UNROLL, _macro, 0)

    for r in range(3):
        scat_wait(r)

    @pl.when(s < CHUNKS - CPT * NS)
    def _():
        idx_start(0, CPT)
        idx_wait(0)
        gather_start(0, 0)
        gather_wait(0, 0)
        compute(0, 0)
        scat_start(0)
        scat_wait(0)

    plsc.subcore_barrier()

    osel = out3.at[c]
    pltpu.sync_copy(acc.at[pl.ds(s * RPT, RPT)], osel.at[pl.ds(s * RPT, RPT)])

    @pl.when(s == 0)
    def _():
        pltpu.sync_copy(acc.at[pl.ds(RPT * NS, TAIL)],
                        osel.at[pl.ds(RPT * NS, TAIL)])


_spmm = pl.kernel(
    _spmm_body,
    out_type=jax.ShapeDtypeStruct((2, N, DH), jnp.float32),
    mesh=_mesh,
    scratch_types=[
        pltpu.VMEM_SHARED((N, DH), jnp.float32),
        pltpu.VMEM((2, K), jnp.int32),
        pltpu.VMEM((2, K), jnp.int32),
        pltpu.VMEM((K,), jnp.float32),
        pltpu.VMEM((K,), jnp.float32),
        pltpu.VMEM((K, DH), jnp.float32),
        pltpu.VMEM((K, DH), jnp.float32),
        pltpu.VMEM((K, DH), jnp.float32),
        pltpu.VMEM((K,), jnp.int32),
        pltpu.VMEM((K,), jnp.int32),
        pltpu.VMEM((K,), jnp.int32),
        pltpu.SemaphoreType.DMA,
        pltpu.SemaphoreType.DMA,
        pltpu.SemaphoreType.DMA,
        pltpu.SemaphoreType.DMA,
        pltpu.SemaphoreType.DMA,
        pltpu.SemaphoreType.DMA,
        pltpu.SemaphoreType.DMA,
        pltpu.SemaphoreType.DMA,
    ],
)


RB = 1000
GRID = N // RB


def _l2n(x):
    sq = jnp.sum(x * x, axis=1, keepdims=True)
    return x * lax.rsqrt(jnp.maximum(sq, 1e-12))


def _pre_body(x_ref, sc_ref, sh_ref, h3_ref, xn_ref):
    x = x_ref[...]
    h = x * sc_ref[...] + sh_ref[...]
    h3_ref[0] = h[:, :DH]
    h3_ref[1] = h[:, DH:]
    xn_ref[...] = _l2n(x)


def _mid_body(ax3_ref, h3_ref, w_ref, b_ref, c_ref,
              sc_ref, sh_ref, yn_ref, h23_ref):
    ax = jnp.concatenate([ax3_ref[0], ax3_ref[1]], axis=1)
    h = jnp.concatenate([h3_ref[0], h3_ref[1]], axis=1)
    z = ax + h * (c_ref[0, 0] + 1.0)
    y = jnp.tanh(jnp.dot(z, w_ref[...], preferred_element_type=jnp.float32)
                 + b_ref[...])
    yn_ref[...] = _l2n(y)
    h2 = y * sc_ref[...] + sh_ref[...]
    h23_ref[0] = h2[:, :DH]
    h23_ref[1] = h2[:, DH:]


def _post_body(ax3_ref, h3_ref, w_ref, b_ref, c_ref,
               xn_ref, y1n_ref, out_ref):
    ax = jnp.concatenate([ax3_ref[0], ax3_ref[1]], axis=1)
    h = jnp.concatenate([h3_ref[0], h3_ref[1]], axis=1)
    z = ax + h * (c_ref[0, 0] + 1.0)
    y2 = jnp.tanh(jnp.dot(z, w_ref[...], preferred_element_type=jnp.float32)
                  + b_ref[...])
    y2n = _l2n(y2)
    xn = xn_ref[...]
    y1n = y1n_ref[...]
    ssum = (jnp.sum(xn * xn, axis=1, keepdims=True)
            + jnp.sum(y1n * y1n, axis=1, keepdims=True)
            + jnp.sum(y2n * y2n, axis=1, keepdims=True))
    r = lax.rsqrt(jnp.maximum(ssum, 1e-12))
    out_ref[:, :D] = xn * r
    out_ref[:, D:2 * D] = y1n * r
    out_ref[:, 2 * D:] = y2n * r


def _row_spec(w):
    return pl.BlockSpec((RB, w), lambda i: (i, 0))


def _half_spec():
    return pl.BlockSpec((2, RB, DH), lambda i: (0, i, 0))


def _rep_spec(shape):
    return pl.BlockSpec(shape, lambda i: tuple(0 for _ in shape))


_smem_spec = pl.BlockSpec(memory_space=pltpu.SMEM)

_h3_shape = jax.ShapeDtypeStruct((2, N, DH), jnp.float32)

_pre = pl.pallas_call(
    _pre_body,
    grid=(GRID,),
    in_specs=[_row_spec(D), _rep_spec((1, D)), _rep_spec((1, D))],
    out_specs=(_half_spec(), _row_spec(D)),
    out_shape=(_h3_shape, jax.ShapeDtypeStruct((N, D), jnp.float32)),
)

_mid = pl.pallas_call(
    _mid_body,
    grid=(GRID,),
    in_specs=[_half_spec(), _half_spec(),
              _rep_spec((D, D)), _rep_spec((1, D)), _smem_spec,
              _rep_spec((1, D)), _rep_spec((1, D))],
    out_specs=(_row_spec(D), _half_spec()),
    out_shape=(jax.ShapeDtypeStruct((N, D), jnp.float32), _h3_shape),
)

_post = pl.pallas_call(
    _post_body,
    grid=(GRID,),
    in_specs=[_half_spec(), _half_spec(),
              _rep_spec((D, D)), _rep_spec((1, D)), _smem_spec,
              _row_spec(D), _row_spec(D)],
    out_specs=_row_spec(3 * D),
    out_shape=jax.ShapeDtypeStruct((N, 3 * D), jnp.float32),
)


@jax.jit
def kernel(x, edge_index, A_values, gamma1, beta1, mean1, var1, w1, b1, c1,
           gamma2, beta2, mean2, var2, w2, b2, c2):
    epk = edge_index

    def _affine(gamma, beta, mean, var):
        s = gamma * lax.rsqrt(var + 1e-3)
        return s.reshape(1, D), (beta - mean * s).reshape(1, D)

    sc1, sh1 = _affine(gamma1, beta1, mean1, var1)
    sc2, sh2 = _affine(gamma2, beta2, mean2, var2)

    h1, xn = _pre(x, sc1, sh1)
    ax1 = _spmm(epk, A_values, h1)
    y1n, h2 = _mid(ax1, h1, w1, b1.reshape(1, D), c1, sc2, sh2)
    ax2 = _spmm(epk, A_values, h2)
    out = _post(ax2, h2, w2, b2.reshape(1, D), c2, xn, y1n)
    return out


# speedup vs baseline: 7.2224x; 1.0249x over previous
import functools

import jax
import jax.numpy as jnp
import numpy as np
from jax import lax
from jax.experimental import pallas as pl
from jax.experimental.pallas import tpu as pltpu
from jax.experimental.pallas import tpu_sc as plsc

N, D, E = 10000, 256, 160000
DH = D // 2
NS = 16
K = 128
CHUNKS = E // K
CPT = CHUNKS // NS
RPT = 624
TAIL = N - RPT * NS

_mesh = plsc.VectorSubcoreMesh(core_axis_name="c", subcore_axis_name="s")

_GDN = lax.GatherDimensionNumbers(offset_dims=(), collapsed_slice_dims=(0,),
                                  start_index_map=(0,))


def _lane_splat(v16, j):
    idx = jnp.full((16, 1), j, jnp.int32)
    return lax.gather(v16, idx, dimension_numbers=_GDN, slice_sizes=(1,),
                      mode=lax.GatherScatterMode.PROMISE_IN_BOUNDS)


def _spmm_body(epk_hbm, val_hbm, h3, out3,
               acc, idx0, idx1, val0, val1, rows0, rows1, rows2,
               dst0, dst1, dst2,
               isem0, isem1, gsem0, gsem1, gsem2, ssem0, ssem1, ssem2):
    c = lax.axis_index("c")
    s = lax.axis_index("s")
    hsel = h3.at[c]
    idxb = (idx0, idx1)
    valb = (val0, val1)
    rowsb = (rows0, rows1, rows2)
    dstb = (dst0, dst1, dst2)
    isem = (isem0, isem1)
    gsem = (gsem0, gsem1, gsem2)
    ssem = (ssem0, ssem1, ssem2)

    zeros16 = jnp.zeros((16,), jnp.float32)

    def _zrow(r, carry):
        for cc in range(DH // 16):
            rows0[r, pl.ds(cc * 16, 16)] = zeros16
        return carry

    lax.fori_loop(0, K, _zrow, 0)
    for j in range(4):
        pltpu.sync_copy(rows0, acc.at[pl.ds(s * RPT + j * K, K)])
    pltpu.sync_copy(rows0.at[pl.ds(0, RPT - 4 * K)],
                    acc.at[pl.ds(s * RPT + 4 * K, RPT - 4 * K)])

    @pl.when(s == 0)
    def _():
        pltpu.sync_copy(rows0.at[pl.ds(0, TAIL)], acc.at[pl.ds(RPT * NS, TAIL)])

    plsc.subcore_barrier()

    def _base(chunk):
        return (s + chunk * NS) * K

    def idx_start(q, chunk):
        base = _base(chunk)
        pltpu.async_copy(epk_hbm.at[:, pl.ds(base, K)], idxb[q], isem[q])
        pltpu.async_copy(val_hbm.at[pl.ds(base, K)], valb[q], isem[q])

    def idx_wait(q):
        pltpu.make_async_copy(epk_hbm.at[:, pl.ds(0, K)], idxb[q], isem[q]).wait()
        pltpu.make_async_copy(val_hbm.at[pl.ds(0, K)], valb[q], isem[q]).wait()

    def gather_start(q, r):
        pltpu.async_copy(hsel.at[idxb[q].at[0]], rowsb[r], gsem[r])

    def gather_wait(q, r):
        pltpu.make_async_copy(hsel.at[idxb[q].at[0]], rowsb[r], gsem[r]).wait()

    def compute(q, r):
        def _group(g, carry):
            sl16 = pl.ds(g * 16, 16)
            dstb[r][sl16] = idxb[q][1, sl16]
            v16 = valb[q][sl16]
            for j in range(16):
                e = g * 16 + j
                vj = _lane_splat(v16, j)
                for cc in range(DH // 16):
                    sl = pl.ds(cc * 16, 16)
                    rowsb[r][e, sl] = rowsb[r][e, sl] * vj
            return carry

        lax.fori_loop(0, K // 16, _group, 0)

    def scat_start(r):
        pltpu.async_copy(rowsb[r], acc.at[dstb[r]], ssem[r], add=True)

    def scat_wait(r):
        pltpu.make_async_copy(rowsb[r], acc.at[dstb[r]], ssem[r]).wait()

    idx_start(0, 0)
    idx_wait(0)
    gather_start(0, 0)
    idx_start(1, 1)

    UNROLL = 6

    def _macro(m, carry):
        for u in range(UNROLL):
            i = m * UNROLL + u
            q = u % 2
            r = u % 3
            nq = (u + 1) % 2
            nr = (u + 1) % 3
            chunk = i

            gather_wait(q, r)

            @pl.when(chunk < CPT - 1)
            def _():
                idx_wait(nq)
                @pl.when(chunk >= 2)
                def _():
                    scat_wait(nr)
                gather_start(nq, nr)

            compute(q, r)
            scat_start(r)

            @pl.when(chunk < CPT - 2)
            def _():
                idx_start(q, chunk + 2)
        return carry

    lax.fori_loop(0, CPT // UNROLL, _macro, 0)

    for r in range(3):
        scat_wait(r)

    @pl.when(s < CHUNKS - CPT * NS)
    def _():
        idx_start(0, CPT)
        idx_wait(0)
        gather_start(0, 0)
        gather_wait(0, 0)
        compute(0, 0)
        scat_start(0)
        scat_wait(0)

    plsc.subcore_barrier()

    osel = out3.at[c]
    pltpu.sync_copy(acc.at[pl.ds(s * RPT, RPT)], osel.at[pl.ds(s * RPT, RPT)])

    @pl.when(s == 0)
    def _():
        pltpu.sync_copy(acc.at[pl.ds(RPT * NS, TAIL)],
                        osel.at[pl.ds(RPT * NS, TAIL)])


_spmm = pl.kernel(
    _spmm_body,
    out_type=jax.ShapeDtypeStruct((2, N, DH), jnp.float32),
    mesh=_mesh,
    scratch_types=[
        pltpu.VMEM_SHARED((N, DH), jnp.float32),
        pltpu.VMEM((2, K), jnp.int32),
        pltpu.VMEM((2, K), jnp.int32),
        pltpu.VMEM((K,), jnp.float32),
        pltpu.VMEM((K,), jnp.float32),
        pltpu.VMEM((K, DH), jnp.float32),
        pltpu.VMEM((K, DH), jnp.float32),
        pltpu.VMEM((K, DH), jnp.float32),
        pltpu.VMEM((K,), jnp.int32),
        pltpu.VMEM((K,), jnp.int32),
        pltpu.VMEM((K,), jnp.int32),
        pltpu.SemaphoreType.DMA,
        pltpu.SemaphoreType.DMA,
        pltpu.SemaphoreType.DMA,
        pltpu.SemaphoreType.DMA,
        pltpu.SemaphoreType.DMA,
        pltpu.SemaphoreType.DMA,
        pltpu.SemaphoreType.DMA,
        pltpu.SemaphoreType.DMA,
    ],
)


RB = 1000
GRID = N // RB


def _l2n(x):
    sq = jnp.sum(x * x, axis=1, keepdims=True)
    return x * lax.rsqrt(jnp.maximum(sq, 1e-12))


def _pre_body(x_ref, sc_ref, sh_ref, h3_ref, xn_ref):
    x = x_ref[...]
    h = x * sc_ref[...] + sh_ref[...]
    h3_ref[0] = h[:, :DH]
    h3_ref[1] = h[:, DH:]
    xn_ref[...] = _l2n(x)


def _mid_body(ax3_ref, h3_ref, w_ref, b_ref, c_ref,
              sc_ref, sh_ref, yn_ref, h23_ref):
    ax = jnp.concatenate([ax3_ref[0], ax3_ref[1]], axis=1)
    h = jnp.concatenate([h3_ref[0], h3_ref[1]], axis=1)
    z = ax + h * (c_ref[0, 0] + 1.0)
    y = jnp.tanh(jnp.dot(z, w_ref[...], preferred_element_type=jnp.float32)
                 + b_ref[...])
    yn_ref[...] = _l2n(y)
    h2 = y * sc_ref[...] + sh_ref[...]
    h23_ref[0] = h2[:, :DH]
    h23_ref[1] = h2[:, DH:]


def _post_body(ax3_ref, h3_ref, w_ref, b_ref, c_ref,
               xn_ref, y1n_ref, out_ref):
    ax = jnp.concatenate([ax3_ref[0], ax3_ref[1]], axis=1)
    h = jnp.concatenate([h3_ref[0], h3_ref[1]], axis=1)
    z = ax + h * (c_ref[0, 0] + 1.0)
    y2 = jnp.tanh(jnp.dot(z, w_ref[...], preferred_element_type=jnp.float32)
                  + b_ref[...])
    y2n = _l2n(y2)
    xn = xn_ref[...]
    y1n = y1n_ref[...]
    ssum = (jnp.sum(xn * xn, axis=1, keepdims=True)
            + jnp.sum(y1n * y1n, axis=1, keepdims=True)
            + jnp.sum(y2n * y2n, axis=1, keepdims=True))
    r = lax.rsqrt(jnp.maximum(ssum, 1e-12))
    out_ref[:, :D] = xn * r
    out_ref[:, D:2 * D] = y1n * r
    out_ref[:, 2 * D:] = y2n * r


def _row_spec(w):
    return pl.BlockSpec((RB, w), lambda i: (i, 0))


def _half_spec():
    return pl.BlockSpec((2, RB, DH), lambda i: (0, i, 0))


def _rep_spec(shape):
    return pl.BlockSpec(shape, lambda i: tuple(0 for _ in shape))


_smem_spec = pl.BlockSpec(memory_space=pltpu.SMEM)

_h3_shape = jax.ShapeDtypeStruct((2, N, DH), jnp.float32)

_pre = pl.pallas_call(
    _pre_body,
    grid=(GRID,),
    in_specs=[_row_spec(D), _rep_spec((1, D)), _rep_spec((1, D))],
    out_specs=(_half_spec(), _row_spec(D)),
    out_shape=(_h3_shape, jax.ShapeDtypeStruct((N, D), jnp.float32)),
)

_mid = pl.pallas_call(
    _mid_body,
    grid=(GRID,),
    in_specs=[_half_spec(), _half_spec(),
              _rep_spec((D, D)), _rep_spec((1, D)), _smem_spec,
              _rep_spec((1, D)), _rep_spec((1, D))],
    out_specs=(_row_spec(D), _half_spec()),
    out_shape=(jax.ShapeDtypeStruct((N, D), jnp.float32), _h3_shape),
)

_post = pl.pallas_call(
    _post_body,
    grid=(GRID,),
    in_specs=[_half_spec(), _half_spec(),
              _rep_spec((D, D)), _rep_spec((1, D)), _smem_spec,
              _row_spec(D), _row_spec(D)],
    out_specs=_row_spec(3 * D),
    out_shape=jax.ShapeDtypeStruct((N, 3 * D), jnp.float32),
)


@jax.jit
def kernel(x, edge_index, A_values, gamma1, beta1, mean1, var1, w1, b1, c1,
           gamma2, beta2, mean2, var2, w2, b2, c2):
    epk = edge_index

    def _affine(gamma, beta, mean, var):
        s = gamma * lax.rsqrt(var + 1e-3)
        return s.reshape(1, D), (beta - mean * s).reshape(1, D)

    sc1, sh1 = _affine(gamma1, beta1, mean1, var1)
    sc2, sh2 = _affine(gamma2, beta2, mean2, var2)

    h1, xn = _pre(x, sc1, sh1)
    ax1 = _spmm(epk, A_values, h1)
    y1n, h2 = _mid(ax1, h1, w1, b1.reshape(1, D), c1, sc2, sh2)
    ax2 = _spmm(epk, A_values, h2)
    out = _post(ax2, h2, w2, b2.reshape(1, D), c2, xn, y1n)
    return out


# speedup vs baseline: 8.0659x; 1.1168x over previous
import functools

import jax
import jax.numpy as jnp
import numpy as np
from jax import lax
from jax.experimental import pallas as pl
from jax.experimental.pallas import tpu as pltpu
from jax.experimental.pallas import tpu_sc as plsc

N, D, E = 10000, 256, 160000
DH = D // 2
NS = 16
K = 128
CHUNKS = E // K
CPT = CHUNKS // NS
RPT = 624
TAIL = N - RPT * NS

_mesh = plsc.VectorSubcoreMesh(core_axis_name="c", subcore_axis_name="s")

_GDN = lax.GatherDimensionNumbers(offset_dims=(), collapsed_slice_dims=(0,),
                                  start_index_map=(0,))


def _lane_splat(v16, j):
    idx = jnp.full((16, 1), j, jnp.int32)
    return lax.gather(v16, idx, dimension_numbers=_GDN, slice_sizes=(1,),
                      mode=lax.GatherScatterMode.PROMISE_IN_BOUNDS)


def _spmm_body(epk_hbm, val_hbm, h3, out3,
               acc, idx0, idx1, idx2, val0, val1, val2, rows0, rows1, rows2,
               dst0, dst1, dst2,
               isem0, isem1, isem2, gsem0, gsem1, gsem2,
               ssem0, ssem1, ssem2):
    c = lax.axis_index("c")
    s = lax.axis_index("s")
    hsel = h3.at[c]
    idxb = (idx0, idx1, idx2)
    valb = (val0, val1, val2)
    rowsb = (rows0, rows1, rows2)
    dstb = (dst0, dst1, dst2)
    isem = (isem0, isem1, isem2)
    gsem = (gsem0, gsem1, gsem2)
    ssem = (ssem0, ssem1, ssem2)


    def _base(chunk):
        return (s + chunk * NS) * K

    def idx_start(q, chunk):
        base = _base(chunk)
        pltpu.async_copy(epk_hbm.at[:, pl.ds(base, K)], idxb[q], isem[q])
        pltpu.async_copy(val_hbm.at[pl.ds(base, K)], valb[q], isem[q])

    def idx_wait(q):
        pltpu.make_async_copy(epk_hbm.at[:, pl.ds(0, K)], idxb[q], isem[q]).wait()
        pltpu.make_async_copy(val_hbm.at[pl.ds(0, K)], valb[q], isem[q]).wait()

    def gather_start(q, r):
        pltpu.async_copy(hsel.at[idxb[q].at[0]], rowsb[r], gsem[r])

    def gather_wait(q, r):
        pltpu.make_async_copy(hsel.at[idxb[q].at[0]], rowsb[r], gsem[r]).wait()

    def compute(q, r):
        def _group(g, carry):
            sl16 = pl.ds(g * 16, 16)
            dstb[r][sl16] = idxb[q][1, sl16]
            v16 = valb[q][sl16]
            for j in range(16):
                e = g * 16 + j
                vj = _lane_splat(v16, j)
                for cc in range(DH // 16):
                    sl = pl.ds(cc * 16, 16)
                    rowsb[r][e, sl] = rowsb[r][e, sl] * vj
            return carry

        lax.fori_loop(0, K // 16, _group, 0)

    def scat_start(r):
        pltpu.async_copy(rowsb[r], acc.at[dstb[r]], ssem[r], add=True)

    def scat_wait(r):
        pltpu.make_async_copy(rowsb[r], acc.at[dstb[r]], ssem[r]).wait()

    idx_start(0, 0)
    idx_start(1, 1)
    idx_start(2, 2)
    idx_wait(0)
    gather_start(0, 0)
    idx_wait(1)
    gather_start(1, 1)

    zeros16 = jnp.zeros((16,), jnp.float32)

    def _zrow(r, carry):
        for cc in range(DH // 16):
            rows2[r, pl.ds(cc * 16, 16)] = zeros16
        return carry

    lax.fori_loop(0, K, _zrow, 0)
    for j in range(4):
        pltpu.sync_copy(rows2, acc.at[pl.ds(s * RPT + j * K, K)])
    pltpu.sync_copy(rows2.at[pl.ds(0, RPT - 4 * K)],
                    acc.at[pl.ds(s * RPT + 4 * K, RPT - 4 * K)])

    @pl.when(s == 0)
    def _():
        pltpu.sync_copy(rows2.at[pl.ds(0, TAIL)], acc.at[pl.ds(RPT * NS, TAIL)])

    plsc.subcore_barrier()

    UNROLL = 3

    def _macro(m, carry):
        for u in range(UNROLL):
            i = m * UNROLL + u
            t = u % 3
            nt = (u + 2) % 3
            chunk = i

            gather_wait(t, t)
            compute(t, t)
            scat_start(t)

            @pl.when(chunk + 3 < CPT)
            def _():
                idx_start(t, chunk + 3)

            @pl.when(chunk + 2 < CPT)
            def _():
                idx_wait(nt)
                @pl.when(chunk >= 1)
                def _():
                    scat_wait(nt)
                gather_start(nt, nt)
        return carry

    lax.fori_loop(0, CPT // UNROLL, _macro, 0)

    for r in range(3):
        scat_wait(r)

    @pl.when(s < CHUNKS - CPT * NS)
    def _():
        idx_start(0, CPT)
        idx_wait(0)
        gather_start(0, 0)
        gather_wait(0, 0)
        compute(0, 0)
        scat_start(0)
        scat_wait(0)

    plsc.subcore_barrier()

    osel = out3.at[c]
    pltpu.sync_copy(acc.at[pl.ds(s * RPT, RPT)], osel.at[pl.ds(s * RPT, RPT)])

    @pl.when(s == 0)
    def _():
        pltpu.sync_copy(acc.at[pl.ds(RPT * NS, TAIL)],
                        osel.at[pl.ds(RPT * NS, TAIL)])


_spmm = pl.kernel(
    _spmm_body,
    out_type=jax.ShapeDtypeStruct((2, N, DH), jnp.float32),
    mesh=_mesh,
    scratch_types=[
        pltpu.VMEM_SHARED((N, DH), jnp.float32),
        pltpu.VMEM((2, K), jnp.int32),
        pltpu.VMEM((2, K), jnp.int32),
        pltpu.VMEM((2, K), jnp.int32),
        pltpu.VMEM((K,), jnp.float32),
        pltpu.VMEM((K,), jnp.float32),
        pltpu.VMEM((K,), jnp.float32),
        pltpu.VMEM((K, DH), jnp.float32),
        pltpu.VMEM((K, DH), jnp.float32),
        pltpu.VMEM((K, DH), jnp.float32),
        pltpu.VMEM((K,), jnp.int32),
        pltpu.VMEM((K,), jnp.int32),
        pltpu.VMEM((K,), jnp.int32),
        pltpu.SemaphoreType.DMA,
        pltpu.SemaphoreType.DMA,
        pltpu.SemaphoreType.DMA,
        pltpu.SemaphoreType.DMA,
        pltpu.SemaphoreType.DMA,
        pltpu.SemaphoreType.DMA,
        pltpu.SemaphoreType.DMA,
        pltpu.SemaphoreType.DMA,
        pltpu.SemaphoreType.DMA,
    ],
)


RB = 1000
GRID = N // RB


def _l2n(x):
    sq = jnp.sum(x * x, axis=1, keepdims=True)
    return x * lax.rsqrt(jnp.maximum(sq, 1e-12))


def _pre_body(x_ref, sc_ref, sh_ref, h3_ref):
    x = x_ref[...]
    h = x * sc_ref[...] + sh_ref[...]
    h3_ref[0] = h[:, :DH]
    h3_ref[1] = h[:, DH:]


def _l2n_body(x_ref, xn_ref):
    xn_ref[...] = _l2n(x_ref[...])


def _mid_body(ax3_ref, h3_ref, w_ref, b_ref, c_ref,
              sc_ref, sh_ref, y_ref, h23_ref):
    ax = jnp.concatenate([ax3_ref[0], ax3_ref[1]], axis=1)
    h = jnp.concatenate([h3_ref[0], h3_ref[1]], axis=1)
    z = ax + h * (c_ref[0, 0] + 1.0)
    y = jnp.tanh(jnp.dot(z, w_ref[...], preferred_element_type=jnp.float32)
                 + b_ref[...])
    y_ref[...] = y
    h2 = y * sc_ref[...] + sh_ref[...]
    h23_ref[0] = h2[:, :DH]
    h23_ref[1] = h2[:, DH:]


def _post_body(ax3_ref, h3_ref, w_ref, b_ref, c_ref,
               xn_ref, y1n_ref, out_ref):
    ax = jnp.concatenate([ax3_ref[0], ax3_ref[1]], axis=1)
    h = jnp.concatenate([h3_ref[0], h3_ref[1]], axis=1)
    z = ax + h * (c_ref[0, 0] + 1.0)
    y2 = jnp.tanh(jnp.dot(z, w_ref[...], preferred_element_type=jnp.float32)
                  + b_ref[...])
    y2n = _l2n(y2)
    xn = xn_ref[...]
    y1n = y1n_ref[...]
    ssum = (jnp.sum(xn * xn, axis=1, keepdims=True)
            + jnp.sum(y1n * y1n, axis=1, keepdims=True)
            + jnp.sum(y2n * y2n, axis=1, keepdims=True))
    r = lax.rsqrt(jnp.maximum(ssum, 1e-12))
    out_ref[:, :D] = xn * r
    out_ref[:, D:2 * D] = y1n * r
    out_ref[:, 2 * D:] = y2n * r


def _row_spec(w):
    return pl.BlockSpec((RB, w), lambda i: (i, 0))


def _half_spec():
    return pl.BlockSpec((2, RB, DH), lambda i: (0, i, 0))


def _rep_spec(shape):
    return pl.BlockSpec(shape, lambda i: tuple(0 for _ in shape))


_smem_spec = pl.BlockSpec(memory_space=pltpu.SMEM)

_h3_shape = jax.ShapeDtypeStruct((2, N, DH), jnp.float32)

_pre = pl.pallas_call(
    _pre_body,
    grid=(GRID,),
    in_specs=[_row_spec(D), _rep_spec((1, D)), _rep_spec((1, D))],
    out_specs=_half_spec(),
    out_shape=_h3_shape,
)

_l2norm = pl.pallas_call(
    _l2n_body,
    grid=(GRID,),
    in_specs=[_row_spec(D)],
    out_specs=_row_spec(D),
    out_shape=jax.ShapeDtypeStruct((N, D), jnp.float32),
)

_mid = pl.pallas_call(
    _mid_body,
    grid=(GRID,),
    in_specs=[_half_spec(), _half_spec(),
              _rep_spec((D, D)), _rep_spec((1, D)), _smem_spec,
              _rep_spec((1, D)), _rep_spec((1, D))],
    out_specs=(_row_spec(D), _half_spec()),
    out_shape=(jax.ShapeDtypeStruct((N, D), jnp.float32), _h3_shape),
)

_post = pl.pallas_call(
    _post_body,
    grid=(GRID,),
    in_specs=[_half_spec(), _half_spec(),
              _rep_spec((D, D)), _rep_spec((1, D)), _smem_spec,
              _row_spec(D), _row_spec(D)],
    out_specs=_row_spec(3 * D),
    out_shape=jax.ShapeDtypeStruct((N, 3 * D), jnp.float32),
)


@jax.jit
def kernel(x, edge_index, A_values, gamma1, beta1, mean1, var1, w1, b1, c1,
           gamma2, beta2, mean2, var2, w2, b2, c2):
    epk = edge_index

    def _affine(gamma, beta, mean, var):
        s = gamma * lax.rsqrt(var + 1e-3)
        return s.reshape(1, D), (beta - mean * s).reshape(1, D)

    sc1, sh1 = _affine(gamma1, beta1, mean1, var1)
    sc2, sh2 = _affine(gamma2, beta2, mean2, var2)

    h1 = _pre(x, sc1, sh1)
    ax1 = _spmm(epk, A_values, h1)
    xn = _l2norm(x)
    y1, h2 = _mid(ax1, h1, w1, b1.reshape(1, D), c1, sc2, sh2)
    ax2 = _spmm(epk, A_values, h2)
    y1n = _l2norm(y1)
    out = _post(ax2, h2, w2, b2.reshape(1, D), c2, xn, y1n)
    return out


# speedup vs baseline: 8.1967x; 1.0162x over previous
import functools

import jax
import jax.numpy as jnp
import numpy as np
from jax import lax
from jax.experimental import pallas as pl
from jax.experimental.pallas import tpu as pltpu
from jax.experimental.pallas import tpu_sc as plsc

N, D, E = 10000, 256, 160000
DH = D // 2
NS = 16
K = 128
CHUNKS = E // K
CPT = CHUNKS // NS
RPT = 624
TAIL = N - RPT * NS

_mesh = plsc.VectorSubcoreMesh(core_axis_name="c", subcore_axis_name="s")

_GDN = lax.GatherDimensionNumbers(offset_dims=(), collapsed_slice_dims=(0,),
                                  start_index_map=(0,))


def _lane_splat(v16, j):
    idx = jnp.full((16, 1), j, jnp.int32)
    return lax.gather(v16, idx, dimension_numbers=_GDN, slice_sizes=(1,),
                      mode=lax.GatherScatterMode.PROMISE_IN_BOUNDS)


def _spmm_body(epk_hbm, val_hbm, h3, out3,
               acc, idx0, idx1, idx2, val0, val1, val2, rows0, rows1, rows2,
               dst0, dst1, dst2,
               isem0, isem1, isem2, gsem0, gsem1, gsem2,
               ssem0, ssem1, ssem2):
    c = lax.axis_index("c")
    s = lax.axis_index("s")
    hsel = h3.at[c]
    idxb = (idx0, idx1, idx2)
    valb = (val0, val1, val2)
    rowsb = (rows0, rows1, rows2)
    dstb = (dst0, dst1, dst2)
    isem = (isem0, isem1, isem2)
    gsem = (gsem0, gsem1, gsem2)
    ssem = (ssem0, ssem1, ssem2)


    def _base(chunk):
        return (s + chunk * NS) * K

    def idx_start(q, chunk):
        base = _base(chunk)
        pltpu.async_copy(epk_hbm.at[:, pl.ds(base, K)], idxb[q], isem[q])
        pltpu.async_copy(val_hbm.at[pl.ds(base, K)], valb[q], isem[q])

    def idx_wait(q):
        pltpu.make_async_copy(epk_hbm.at[:, pl.ds(0, K)], idxb[q], isem[q]).wait()
        pltpu.make_async_copy(val_hbm.at[pl.ds(0, K)], valb[q], isem[q]).wait()

    def gather_start(q, r):
        pltpu.async_copy(hsel.at[idxb[q].at[0]], rowsb[r], gsem[r])

    def gather_wait(q, r):
        pltpu.make_async_copy(hsel.at[idxb[q].at[0]], rowsb[r], gsem[r]).wait()

    def compute(q, r):
        def _group(g, carry):
            sl16 = pl.ds(g * 16, 16)
            dstb[r][sl16] = idxb[q][1, sl16]
            v16 = valb[q][sl16]
            for j in range(16):
                e = g * 16 + j
                vj = _lane_splat(v16, j)
                for cc in range(DH // 16):
                    sl = pl.ds(cc * 16, 16)
                    rowsb[r][e, sl] = rowsb[r][e, sl] * vj
            return carry

        lax.fori_loop(0, K // 16, _group, 0)

    def scat_start(r):
        pltpu.async_copy(rowsb[r], acc.at[dstb[r]], ssem[r], add=True)

    def scat_wait(r):
        pltpu.make_async_copy(rowsb[r], acc.at[dstb[r]], ssem[r]).wait()

    idx_start(0, 0)
    idx_start(1, 1)
    idx_start(2, 2)
    idx_wait(0)
    gather_start(0, 0)
    idx_wait(1)
    gather_start(1, 1)

    zeros16 = jnp.zeros((16,), jnp.float32)

    def _zrow(r, carry):
        for cc in range(DH // 16):
            rows2[r, pl.ds(cc * 16, 16)] = zeros16
        return carry

    lax.fori_loop(0, K, _zrow, 0)
    for j in range(4):
        pltpu.sync_copy(rows2, acc.at[pl.ds(s * RPT + j * K, K)])
    pltpu.sync_copy(rows2.at[pl.ds(0, RPT - 4 * K)],
                    acc.at[pl.ds(s * RPT + 4 * K, RPT - 4 * K)])

    @pl.when(s == 0)
    def _():
        pltpu.sync_copy(rows2.at[pl.ds(0, TAIL)], acc.at[pl.ds(RPT * NS, TAIL)])

    plsc.subcore_barrier()

    UNROLL = 3

    my_n = CPT + jnp.where(s < CHUNKS - CPT * NS, 1, 0)
    NMACRO = (CPT + 1 + UNROLL - 1) // UNROLL

    def _macro(m, carry):
        for u in range(UNROLL):
            i = m * UNROLL + u
            t = u % 3
            nt = (u + 2) % 3
            chunk = i

            @pl.when(chunk < my_n)
            def _():
                gather_wait(t, t)
                compute(t, t)
                scat_start(t)

            @pl.when(chunk + 3 < my_n)
            def _():
                idx_start(t, chunk + 3)

            @pl.when(chunk + 2 < my_n)
            def _():
                idx_wait(nt)
                @pl.when(chunk >= 1)
                def _():
                    scat_wait(nt)
                gather_start(nt, nt)
        return carry

    lax.fori_loop(0, NMACRO, _macro, 0)

    for r in range(3):
        scat_wait(r)

    plsc.subcore_barrier()

    osel = out3.at[c]
    pltpu.sync_copy(acc.at[pl.ds(s * RPT, RPT)], osel.at[pl.ds(s * RPT, RPT)])

    @pl.when(s == 0)
    def _():
        pltpu.sync_copy(acc.at[pl.ds(RPT * NS, TAIL)],
                        osel.at[pl.ds(RPT * NS, TAIL)])


_spmm = pl.kernel(
    _spmm_body,
    out_type=jax.ShapeDtypeStruct((2, N, DH), jnp.float32),
    mesh=_mesh,
    scratch_types=[
        pltpu.VMEM_SHARED((N, DH), jnp.float32),
        pltpu.VMEM((2, K), jnp.int32),
        pltpu.VMEM((2, K), jnp.int32),
        pltpu.VMEM((2, K), jnp.int32),
        pltpu.VMEM((K,), jnp.float32),
        pltpu.VMEM((K,), jnp.float32),
        pltpu.VMEM((K,), jnp.float32),
        pltpu.VMEM((K, DH), jnp.float32),
        pltpu.VMEM((K, DH), jnp.float32),
        pltpu.VMEM((K, DH), jnp.float32),
        pltpu.VMEM((K,), jnp.int32),
        pltpu.VMEM((K,), jnp.int32),
        pltpu.VMEM((K,), jnp.int32),
        pltpu.SemaphoreType.DMA,
        pltpu.SemaphoreType.DMA,
        pltpu.SemaphoreType.DMA,
        pltpu.SemaphoreType.DMA,
        pltpu.SemaphoreType.DMA,
        pltpu.SemaphoreType.DMA,
        pltpu.SemaphoreType.DMA,
        pltpu.SemaphoreType.DMA,
        pltpu.SemaphoreType.DMA,
    ],
)


RB = 1000
GRID = N // RB


def _l2n(x):
    sq = jnp.sum(x * x, axis=1, keepdims=True)
    return x * lax.rsqrt(jnp.maximum(sq, 1e-12))


def _pre_body(x_ref, sc_ref, sh_ref, h3_ref):
    x = x_ref[...]
    h = x * sc_ref[...] + sh_ref[...]
    h3_ref[0] = h[:, :DH]
    h3_ref[1] = h[:, DH:]


def _l2n_body(x_ref, xn_ref):
    xn_ref[...] = _l2n(x_ref[...])


def _mid_body(ax3_ref, h3_ref, w_ref, b_ref, c_ref,
              sc_ref, sh_ref, y_ref, h23_ref):
    ax = jnp.concatenate([ax3_ref[0], ax3_ref[1]], axis=1)
    h = jnp.concatenate([h3_ref[0], h3_ref[1]], axis=1)
    z = ax + h * (c_ref[0, 0] + 1.0)
    y = jnp.tanh(jnp.dot(z, w_ref[...], preferred_element_type=jnp.float32)
                 + b_ref[...])
    y_ref[...] = y
    h2 = y * sc_ref[...] + sh_ref[...]
    h23_ref[0] = h2[:, :DH]
    h23_ref[1] = h2[:, DH:]


def _post_body(ax3_ref, h3_ref, w_ref, b_ref, c_ref,
               xn_ref, y1n_ref, out_ref):
    ax = jnp.concatenate([ax3_ref[0], ax3_ref[1]], axis=1)
    h = jnp.concatenate([h3_ref[0], h3_ref[1]], axis=1)
    z = ax + h * (c_ref[0, 0] + 1.0)
    y2 = jnp.tanh(jnp.dot(z, w_ref[...], preferred_element_type=jnp.float32)
                  + b_ref[...])
    y2n = _l2n(y2)
    xn = xn_ref[...]
    y1n = y1n_ref[...]
    ssum = (jnp.sum(xn * xn, axis=1, keepdims=True)
            + jnp.sum(y1n * y1n, axis=1, keepdims=True)
            + jnp.sum(y2n * y2n, axis=1, keepdims=True))
    r = lax.rsqrt(jnp.maximum(ssum, 1e-12))
    out_ref[:, :D] = xn * r
    out_ref[:, D:2 * D] = y1n * r
    out_ref[:, 2 * D:] = y2n * r


def _row_spec(w):
    return pl.BlockSpec((RB, w), lambda i: (i, 0))


def _half_spec():
    return pl.BlockSpec((2, RB, DH), lambda i: (0, i, 0))


def _rep_spec(shape):
    return pl.BlockSpec(shape, lambda i: tuple(0 for _ in shape))


_smem_spec = pl.BlockSpec(memory_space=pltpu.SMEM)

_h3_shape = jax.ShapeDtypeStruct((2, N, DH), jnp.float32)

_pre = pl.pallas_call(
    _pre_body,
    grid=(GRID,),
    in_specs=[_row_spec(D), _rep_spec((1, D)), _rep_spec((1, D))],
    out_specs=_half_spec(),
    out_shape=_h3_shape,
)

_l2norm = pl.pallas_call(
    _l2n_body,
    grid=(GRID,),
    in_specs=[_row_spec(D)],
    out_specs=_row_spec(D),
    out_shape=jax.ShapeDtypeStruct((N, D), jnp.float32),
)

_mid = pl.pallas_call(
    _mid_body,
    grid=(GRID,),
    in_specs=[_half_spec(), _half_spec(),
              _rep_spec((D, D)), _rep_spec((1, D)), _smem_spec,
              _rep_spec((1, D)), _rep_spec((1, D))],
    out_specs=(_row_spec(D), _half_spec()),
    out_shape=(jax.ShapeDtypeStruct((N, D), jnp.float32), _h3_shape),
)

_post = pl.pallas_call(
    _post_body,
    grid=(GRID,),
    in_specs=[_half_spec(), _half_spec(),
              _rep_spec((D, D)), _rep_spec((1, D)), _smem_spec,
              _row_spec(D), _row_spec(D)],
    out_specs=_row_spec(3 * D),
    out_shape=jax.ShapeDtypeStruct((N, 3 * D), jnp.float32),
)


@jax.jit
def kernel(x, edge_index, A_values, gamma1, beta1, mean1, var1, w1, b1, c1,
           gamma2, beta2, mean2, var2, w2, b2, c2):
    epk = edge_index

    def _affine(gamma, beta, mean, var):
        s = gamma * lax.rsqrt(var + 1e-3)
        return s.reshape(1, D), (beta - mean * s).reshape(1, D)

    sc1, sh1 = _affine(gamma1, beta1, mean1, var1)
    sc2, sh2 = _affine(gamma2, beta2, mean2, var2)

    h1 = _pre(x, sc1, sh1)
    ax1 = _spmm(epk, A_values, h1)
    xn = _l2norm(x)
    y1, h2 = _mid(ax1, h1, w1, b1.reshape(1, D), c1, sc2, sh2)
    ax2 = _spmm(epk, A_values, h2)
    y1n = _l2norm(y1)
    out = _post(ax2, h2, w2, b2.reshape(1, D), c2, xn, y1n)
    return out


# speedup vs baseline: 8.2093x; 1.0015x over previous
import functools

import jax
import jax.numpy as jnp
import numpy as np
from jax import lax
from jax.experimental import pallas as pl
from jax.experimental.pallas import tpu as pltpu
from jax.experimental.pallas import tpu_sc as plsc

N, D, E = 10000, 256, 160000
DH = D // 2
NS = 16
K = 128
CHUNKS = E // K
CPT = CHUNKS // NS
RPT = 624
TAIL = N - RPT * NS

_mesh = plsc.VectorSubcoreMesh(core_axis_name="c", subcore_axis_name="s")

_GDN = lax.GatherDimensionNumbers(offset_dims=(), collapsed_slice_dims=(0,),
                                  start_index_map=(0,))


def _lane_splat(v16, j):
    idx = jnp.full((16, 1), j, jnp.int32)
    return lax.gather(v16, idx, dimension_numbers=_GDN, slice_sizes=(1,),
                      mode=lax.GatherScatterMode.PROMISE_IN_BOUNDS)


def _spmm_body(epk_hbm, val_hbm, h3, out3,
               acc, idx0, idx1, idx2, val0, val1, val2, rows0, rows1, rows2,
               dst0, dst1, dst2,
               isem0, isem1, isem2, gsem0, gsem1, gsem2,
               ssem0, ssem1, ssem2):
    c = lax.axis_index("c")
    s = lax.axis_index("s")
    hsel = h3.at[c]
    idxb = (idx0, idx1, idx2)
    valb = (val0, val1, val2)
    rowsb = (rows0, rows1, rows2)
    dstb = (dst0, dst1, dst2)
    isem = (isem0, isem1, isem2)
    gsem = (gsem0, gsem1, gsem2)
    ssem = (ssem0, ssem1, ssem2)


    def _base(chunk):
        return (s + chunk * NS) * K

    def idx_start(q, chunk):
        base = _base(chunk)
        pltpu.async_copy(epk_hbm.at[:, pl.ds(base, K)], idxb[q], isem[q])
        pltpu.async_copy(val_hbm.at[pl.ds(base, K)], valb[q], isem[q])

    def idx_wait(q):
        pltpu.make_async_copy(epk_hbm.at[:, pl.ds(0, K)], idxb[q], isem[q]).wait()
        pltpu.make_async_copy(val_hbm.at[pl.ds(0, K)], valb[q], isem[q]).wait()

    H2 = K // 2

    def gather_start(q, r):
        pltpu.async_copy(hsel.at[idxb[q].at[0, pl.ds(0, H2)]],
                         rowsb[r].at[pl.ds(0, H2)], gsem[r])
        pltpu.async_copy(hsel.at[idxb[q].at[0, pl.ds(H2, H2)]],
                         rowsb[r].at[pl.ds(H2, H2)], gsem[r])

    def gather_wait(q, r):
        pltpu.make_async_copy(hsel.at[idxb[q].at[0, pl.ds(0, H2)]],
                              rowsb[r].at[pl.ds(0, H2)], gsem[r]).wait()
        pltpu.make_async_copy(hsel.at[idxb[q].at[0, pl.ds(H2, H2)]],
                              rowsb[r].at[pl.ds(H2, H2)], gsem[r]).wait()

    def compute(q, r):
        def _group(g, carry):
            sl16 = pl.ds(g * 16, 16)
            dstb[r][sl16] = idxb[q][1, sl16]
            v16 = valb[q][sl16]
            for j in range(16):
                e = g * 16 + j
                vj = _lane_splat(v16, j)
                for cc in range(DH // 16):
                    sl = pl.ds(cc * 16, 16)
                    rowsb[r][e, sl] = rowsb[r][e, sl] * vj
            return carry

        lax.fori_loop(0, K // 16, _group, 0)

    def scat_start(r):
        pltpu.async_copy(rowsb[r], acc.at[dstb[r]], ssem[r], add=True)

    def scat_wait(r):
        pltpu.make_async_copy(rowsb[r], acc.at[dstb[r]], ssem[r]).wait()

    idx_start(0, 0)
    idx_start(1, 1)
    idx_start(2, 2)
    idx_wait(0)
    gather_start(0, 0)
    idx_wait(1)
    gather_start(1, 1)

    zeros16 = jnp.zeros((16,), jnp.float32)

    def _zrow(r, carry):
        for cc in range(DH // 16):
            rows2[r, pl.ds(cc * 16, 16)] = zeros16
        return carry

    lax.fori_loop(0, K, _zrow, 0)
    for j in range(4):
        pltpu.sync_copy(rows2, acc.at[pl.ds(s * RPT + j * K, K)])
    pltpu.sync_copy(rows2.at[pl.ds(0, RPT - 4 * K)],
                    acc.at[pl.ds(s * RPT + 4 * K, RPT - 4 * K)])

    @pl.when(s == 0)
    def _():
        pltpu.sync_copy(rows2.at[pl.ds(0, TAIL)], acc.at[pl.ds(RPT * NS, TAIL)])

    plsc.subcore_barrier()

    UNROLL = 3

    my_n = CPT + jnp.where(s < CHUNKS - CPT * NS, 1, 0)
    NMACRO = (CPT + 1 + UNROLL - 1) // UNROLL

    def _macro(m, carry):
        for u in range(UNROLL):
            i = m * UNROLL + u
            t = u % 3
            nt = (u + 2) % 3
            chunk = i

            @pl.when(chunk < my_n)
            def _():
                gather_wait(t, t)
                compute(t, t)
                scat_start(t)

            @pl.when(chunk + 3 < my_n)
            def _():
                idx_start(t, chunk + 3)

            @pl.when(chunk + 2 < my_n)
            def _():
                idx_wait(nt)
                @pl.when(chunk >= 1)
                def _():
                    scat_wait(nt)
                gather_start(nt, nt)
        return carry

    lax.fori_loop(0, NMACRO, _macro, 0)

    for r in range(3):
        scat_wait(r)

    plsc.subcore_barrier()

    osel = out3.at[c]
    pltpu.sync_copy(acc.at[pl.ds(s * RPT, RPT)], osel.at[pl.ds(s * RPT, RPT)])

    @pl.when(s == 0)
    def _():
        pltpu.sync_copy(acc.at[pl.ds(RPT * NS, TAIL)],
                        osel.at[pl.ds(RPT * NS, TAIL)])


_spmm = pl.kernel(
    _spmm_body,
    out_type=jax.ShapeDtypeStruct((2, N, DH), jnp.float32),
    mesh=_mesh,
    scratch_types=[
        pltpu.VMEM_SHARED((N, DH), jnp.float32),
        pltpu.VMEM((2, K), jnp.int32),
        pltpu.VMEM((2, K), jnp.int32),
        pltpu.VMEM((2, K), jnp.int32),
        pltpu.VMEM((K,), jnp.float32),
        pltpu.VMEM((K,), jnp.float32),
        pltpu.VMEM((K,), jnp.float32),
        pltpu.VMEM((K, DH), jnp.float32),
        pltpu.VMEM((K, DH), jnp.float32),
        pltpu.VMEM((K, DH), jnp.float32),
        pltpu.VMEM((K,), jnp.int32),
        pltpu.VMEM((K,), jnp.int32),
        pltpu.VMEM((K,), jnp.int32),
        pltpu.SemaphoreType.DMA,
        pltpu.SemaphoreType.DMA,
        pltpu.SemaphoreType.DMA,
        pltpu.SemaphoreType.DMA,
        pltpu.SemaphoreType.DMA,
        pltpu.SemaphoreType.DMA,
        pltpu.SemaphoreType.DMA,
        pltpu.SemaphoreType.DMA,
        pltpu.SemaphoreType.DMA,
    ],
)


RB = 1000
GRID = N // RB


def _l2n(x):
    sq = jnp.sum(x * x, axis=1, keepdims=True)
    return x * lax.rsqrt(jnp.maximum(sq, 1e-12))


def _pre_body(x_ref, sc_ref, sh_ref, h3_ref):
    x = x_ref[...]
    h = x * sc_ref[...] + sh_ref[...]
    h3_ref[0] = h[:, :DH]
    h3_ref[1] = h[:, DH:]


def _l2n_body(x_ref, xn_ref):
    xn_ref[...] = _l2n(x_ref[...])


def _mid_body(ax3_ref, h3_ref, w_ref, b_ref, c_ref,
              sc_ref, sh_ref, y_ref, h23_ref):
    ax = jnp.concatenate([ax3_ref[0], ax3_ref[1]], axis=1)
    h = jnp.concatenate([h3_ref[0], h3_ref[1]], axis=1)
    z = ax + h * (c_ref[0, 0] + 1.0)
    y = jnp.tanh(jnp.dot(z, w_ref[...], preferred_element_type=jnp.float32)
                 + b_ref[...])
    y_ref[...] = y
    h2 = y * sc_ref[...] + sh_ref[...]
    h23_ref[0] = h2[:, :DH]
    h23_ref[1] = h2[:, DH:]


def _post_body(ax3_ref, h3_ref, w_ref, b_ref, c_ref,
               xn_ref, y1n_ref, out_ref):
    ax = jnp.concatenate([ax3_ref[0], ax3_ref[1]], axis=1)
    h = jnp.concatenate([h3_ref[0], h3_ref[1]], axis=1)
    z = ax + h * (c_ref[0, 0] + 1.0)
    y2 = jnp.tanh(jnp.dot(z, w_ref[...], preferred_element_type=jnp.float32)
                  + b_ref[...])
    y2n = _l2n(y2)
    xn = xn_ref[...]
    y1n = y1n_ref[...]
    ssum = (jnp.sum(xn * xn, axis=1, keepdims=True)
            + jnp.sum(y1n * y1n, axis=1, keepdims=True)
            + jnp.sum(y2n * y2n, axis=1, keepdims=True))
    r = lax.rsqrt(jnp.maximum(ssum, 1e-12))
    out_ref[:, :D] = xn * r
    out_ref[:, D:2 * D] = y1n * r
    out_ref[:, 2 * D:] = y2n * r


def _row_spec(w):
    return pl.BlockSpec((RB, w), lambda i: (i, 0))


def _half_spec():
    return pl.BlockSpec((2, RB, DH), lambda i: (0, i, 0))


def _rep_spec(shape):
    return pl.BlockSpec(shape, lambda i: tuple(0 for _ in shape))


_smem_spec = pl.BlockSpec(memory_space=pltpu.SMEM)

_h3_shape = jax.ShapeDtypeStruct((2, N, DH), jnp.float32)

_pre = pl.pallas_call(
    _pre_body,
    grid=(GRID,),
    in_specs=[_row_spec(D), _rep_spec((1, D)), _rep_spec((1, D))],
    out_specs=_half_spec(),
    out_shape=_h3_shape,
)

_l2norm = pl.pallas_call(
    _l2n_body,
    grid=(GRID,),
    in_specs=[_row_spec(D)],
    out_specs=_row_spec(D),
    out_shape=jax.ShapeDtypeStruct((N, D), jnp.float32),
)

_mid = pl.pallas_call(
    _mid_body,
    grid=(GRID,),
    in_specs=[_half_spec(), _half_spec(),
              _rep_spec((D, D)), _rep_spec((1, D)), _smem_spec,
              _rep_spec((1, D)), _rep_spec((1, D))],
    out_specs=(_row_spec(D), _half_spec()),
    out_shape=(jax.ShapeDtypeStruct((N, D), jnp.float32), _h3_shape),
)

_post = pl.pallas_call(
    _post_body,
    grid=(GRID,),
    in_specs=[_half_spec(), _half_spec(),
              _rep_spec((D, D)), _rep_spec((1, D)), _smem_spec,
              _row_spec(D), _row_spec(D)],
    out_specs=_row_spec(3 * D),
    out_shape=jax.ShapeDtypeStruct((N, 3 * D), jnp.float32),
)


@jax.jit
def kernel(x, edge_index, A_values, gamma1, beta1, mean1, var1, w1, b1, c1,
           gamma2, beta2, mean2, var2, w2, b2, c2):
    epk = edge_index

    def _affine(gamma, beta, mean, var):
        s = gamma * lax.rsqrt(var + 1e-3)
        return s.reshape(1, D), (beta - mean * s).reshape(1, D)

    sc1, sh1 = _affine(gamma1, beta1, mean1, var1)
    sc2, sh2 = _affine(gamma2, beta2, mean2, var2)

    h1 = _pre(x, sc1, sh1)
    ax1 = _spmm(epk, A_values, h1)
    xn = _l2norm(x)
    y1, h2 = _mid(ax1, h1, w1, b1.reshape(1, D), c1, sc2, sh2)
    ax2 = _spmm(epk, A_values, h2)
    y1n = _l2norm(y1)
    out = _post(ax2, h2, w2, b2.reshape(1, D), c2, xn, y1n)
    return out
